# initial kernel scaffold (unmeasured)
import jax
import jax.numpy as jnp
from jax import lax
from jax.experimental import pallas as pl
from jax.experimental.pallas import tpu as pltpu

N_DEV = 4
T = 1024
T_PER = 256
D = 1024
F = 2048
E = 16
E_PER = 4

_DN = (((1,), (0,)), ((), ()))


def kernel(x, router, W1, W2):
    def body(x_ref, r_ref, w1_ref, w2_ref, out_ref,
             rcomm_ref, wts_ref, wcomm_ref, xg_ref, xcomm_ref,
             acc_ref, w1v_ref, w2v_ref, rscomm_ref,
             r_send, r_recv, w_send, w_recv, x_send, x_recv,
             rs_send, rs_recv, load_sems):
        my = lax.axis_index("i")
        left = lax.rem(my + N_DEV - 1, N_DEV)
        right = lax.rem(my + 1, N_DEV)

        barrier = pltpu.get_barrier_semaphore()
        for nbr in (left, right):
            pl.semaphore_signal(
                barrier, inc=1, device_id=(nbr,),
                device_id_type=pl.DeviceIdType.MESH,
            )
        pl.semaphore_wait(barrier, 2)

        rcomm_ref[0] = r_ref[...]
        xcomm_ref[0] = x_ref[...].astype(jnp.bfloat16)
        for h in range(N_DEV - 1):
            ra = pltpu.make_async_remote_copy(
                src_ref=rcomm_ref.at[h], dst_ref=rcomm_ref.at[h + 1],
                send_sem=r_send.at[h], recv_sem=r_recv.at[h],
                device_id=(right,), device_id_type=pl.DeviceIdType.MESH,
            )
            xa = pltpu.make_async_remote_copy(
                src_ref=xcomm_ref.at[h], dst_ref=xcomm_ref.at[h + 1],
                send_sem=x_send.at[h], recv_sem=x_recv.at[h],
                device_id=(right,), device_id_type=pl.DeviceIdType.MESH,
            )
            ra.start()
            xa.start()
            ra.wait()
            xa.wait()

        for s in range(N_DEV):
            org = lax.rem(my + N_DEV - s, N_DEV)
            xg_ref[pl.ds(org * T_PER, T_PER), :] = xcomm_ref[s]

        lane_d = lax.broadcasted_iota(jnp.int32, (D, E), 1) // E_PER
        rfull = jnp.zeros((D, E), jnp.float32)
        for s in range(N_DEV):
            org = lax.rem(my + N_DEV - s, N_DEV)
            tiled = jnp.concatenate([rcomm_ref[s]] * N_DEV, axis=1)
            rfull = jnp.where(lane_d == org, tiled, rfull)

        g = lax.dot_general(x_ref[...], rfull, _DN,
                            preferred_element_type=jnp.float32)
        lane = lax.broadcasted_iota(jnp.int32, (T_PER, E), 1)
        m1 = jnp.max(g, axis=1, keepdims=True)
        a1 = jnp.min(jnp.where(g >= m1, lane, E), axis=1, keepdims=True)
        oh1 = lane == a1
        gm = jnp.where(oh1, jnp.float32(-1e30), g)
        m2 = jnp.max(gm, axis=1, keepdims=True)
        a2 = jnp.min(jnp.where(gm >= m2, lane, E), axis=1, keepdims=True)
        oh2 = lane == a2
        e2 = jnp.exp(m2 - m1)
        wt1 = 1.0 / (1.0 + e2)
        wt2 = e2 / (1.0 + e2)
        wd = oh1.astype(jnp.float32) * wt1 + oh2.astype(jnp.float32) * wt2

        wcomm_ref[0] = wd
        for h in range(N_DEV - 1):
            wa = pltpu.make_async_remote_copy(
                src_ref=wcomm_ref.at[h], dst_ref=wcomm_ref.at[h + 1],
                send_sem=w_send.at[h], recv_sem=w_recv.at[h],
                device_id=(right,), device_id_type=pl.DeviceIdType.MESH,
            )
            wa.start()
            wa.wait()
        for s in range(N_DEV):
            org = lax.rem(my + N_DEV - s, N_DEV)
            wts_ref[pl.ds(org * T_PER, T_PER), :] = wcomm_ref[s]

        lane_t = lax.broadcasted_iota(jnp.int32, (T, E), 1)
        for e in range(E_PER):
            c1 = pltpu.make_async_copy(w1_ref.at[e], w1v_ref, load_sems.at[0])
            c2 = pltpu.make_async_copy(w2_ref.at[e], w2v_ref, load_sems.at[1])
            c1.start()
            c2.start()
            c1.wait()
            c2.wait()
            h1 = lax.dot_general(
                xg_ref[...], w1v_ref[...].astype(jnp.bfloat16), _DN,
                preferred_element_type=jnp.float32)
            hb = jnp.maximum(h1, 0.0).astype(jnp.bfloat16)
            p = lax.dot_general(
                hb, w2v_ref[...].astype(jnp.bfloat16), _DN,
                preferred_element_type=jnp.float32)
            ge = my * E_PER + e
            col = jnp.sum(
                jnp.where(lane_t == ge, wts_ref[...], 0.0),
                axis=1, keepdims=True)
            if e == 0:
                acc_ref[...] = p * col
            else:
                acc_ref[...] = acc_ref[...] + p * col

        idx0 = lax.rem(my + N_DEV - 1, N_DEV)
        rscomm_ref[0] = acc_ref[pl.ds(idx0 * T_PER, T_PER), :].astype(
            jnp.bfloat16)
        for h in range(N_DEV - 1):
            rr = pltpu.make_async_remote_copy(
                src_ref=rscomm_ref.at[h], dst_ref=rscomm_ref.at[h + 1],
                send_sem=rs_send.at[h], recv_sem=rs_recv.at[h],
                device_id=(right,), device_id_type=pl.DeviceIdType.MESH,
            )
            rr.start()
            rr.wait()
            idx = lax.rem(my + 2 * N_DEV - 2 - h, N_DEV)
            chunk = acc_ref[pl.ds(idx * T_PER, T_PER), :]
            if h < N_DEV - 2:
                rscomm_ref[h + 1] = (
                    rscomm_ref[h + 1].astype(jnp.float32) + chunk
                ).astype(jnp.bfloat16)
            else:
                out_ref[...] = rscomm_ref[h + 1].astype(jnp.float32) + chunk

    return pl.pallas_call(
        body,
        out_shape=jax.ShapeDtypeStruct((T_PER, D), jnp.float32),
        in_specs=[
            pl.BlockSpec(memory_space=pltpu.VMEM),
            pl.BlockSpec(memory_space=pltpu.VMEM),
            pl.BlockSpec(memory_space=pltpu.ANY),
            pl.BlockSpec(memory_space=pltpu.ANY),
        ],
        out_specs=pl.BlockSpec(memory_space=pltpu.VMEM),
        scratch_shapes=[
            pltpu.VMEM((N_DEV, D, E_PER), jnp.float32),
            pltpu.VMEM((T, E), jnp.float32),
            pltpu.VMEM((N_DEV, T_PER, E), jnp.float32),
            pltpu.VMEM((T, D), jnp.bfloat16),
            pltpu.VMEM((N_DEV, T_PER, D), jnp.bfloat16),
            pltpu.VMEM((T, D), jnp.float32),
            pltpu.VMEM((D, F), jnp.float32),
            pltpu.VMEM((F, D), jnp.float32),
            pltpu.VMEM((N_DEV, T_PER, D), jnp.bfloat16),
            pltpu.SemaphoreType.DMA((N_DEV - 1,)),
            pltpu.SemaphoreType.DMA((N_DEV - 1,)),
            pltpu.SemaphoreType.DMA((N_DEV - 1,)),
            pltpu.SemaphoreType.DMA((N_DEV - 1,)),
            pltpu.SemaphoreType.DMA((N_DEV - 1,)),
            pltpu.SemaphoreType.DMA((N_DEV - 1,)),
            pltpu.SemaphoreType.DMA((N_DEV - 1,)),
            pltpu.SemaphoreType.DMA((N_DEV - 1,)),
            pltpu.SemaphoreType.DMA((2,)),
        ],
        compiler_params=pltpu.CompilerParams(
            collective_id=0,
            vmem_limit_bytes=128 * 1024 * 1024,
        ),
    )(x, router, W1, W2)


# baseline (device time: 145062 ns/iter reference)
import jax
import jax.numpy as jnp
from jax import lax
from jax.experimental import pallas as pl
from jax.experimental.pallas import tpu as pltpu

N_DEV = 4
T = 1024
T_PER = 256
D = 1024
F = 2048
E = 16
E_PER = 4

_DN = (((1,), (0,)), ((), ()))


def kernel(x, router, W1, W2):
    def body(x_ref, r_ref, w1_ref, w2_ref, out_ref,
             rcomm_ref, wts_ref, wcomm_ref, xg_ref, xcomm_ref,
             acc_ref, w1v_ref, w2v_ref, rscomm_ref,
             r_send, r_recv, w_send, w_recv, x_send, x_recv,
             rs_send, rs_recv, load_sems):
        my = lax.axis_index("i")
        left = lax.rem(my + N_DEV - 1, N_DEV)
        right = lax.rem(my + 1, N_DEV)

        barrier = pltpu.get_barrier_semaphore()
        for nbr in (left, right):
            pl.semaphore_signal(
                barrier, inc=1, device_id=(nbr,),
                device_id_type=pl.DeviceIdType.MESH,
            )
        pl.semaphore_wait(barrier, 2)

        rcomm_ref[0] = r_ref[...]
        xcomm_ref[0] = x_ref[...].astype(jnp.bfloat16)
        for h in range(N_DEV - 1):
            ra = pltpu.make_async_remote_copy(
                src_ref=rcomm_ref.at[h], dst_ref=rcomm_ref.at[h + 1],
                send_sem=r_send.at[h], recv_sem=r_recv.at[h],
                device_id=(right,), device_id_type=pl.DeviceIdType.MESH,
            )
            xa = pltpu.make_async_remote_copy(
                src_ref=xcomm_ref.at[h], dst_ref=xcomm_ref.at[h + 1],
                send_sem=x_send.at[h], recv_sem=x_recv.at[h],
                device_id=(right,), device_id_type=pl.DeviceIdType.MESH,
            )
            ra.start()
            xa.start()
            ra.wait()
            xa.wait()

        for s in range(N_DEV):
            org = lax.rem(my + N_DEV - s, N_DEV)
            xg_ref[pl.ds(org * T_PER, T_PER), :] = xcomm_ref[s]

        lane_d = lax.broadcasted_iota(jnp.int32, (D, E), 1) // E_PER
        rfull = jnp.zeros((D, E), jnp.float32)
        for s in range(N_DEV):
            org = lax.rem(my + N_DEV - s, N_DEV)
            tiled = jnp.concatenate([rcomm_ref[s]] * N_DEV, axis=1)
            rfull = jnp.where(lane_d == org, tiled, rfull)

        g = lax.dot_general(x_ref[...], rfull, _DN,
                            precision=lax.Precision.HIGHEST,
                            preferred_element_type=jnp.float32)
        lane = lax.broadcasted_iota(jnp.int32, (T_PER, E), 1)
        m1 = jnp.max(g, axis=1, keepdims=True)
        a1 = jnp.min(jnp.where(g >= m1, lane, E), axis=1, keepdims=True)
        oh1 = lane == a1
        gm = jnp.where(oh1, jnp.float32(-1e30), g)
        m2 = jnp.max(gm, axis=1, keepdims=True)
        a2 = jnp.min(jnp.where(gm >= m2, lane, E), axis=1, keepdims=True)
        oh2 = lane == a2
        e2 = jnp.exp(m2 - m1)
        wt1 = 1.0 / (1.0 + e2)
        wt2 = e2 / (1.0 + e2)
        wd = oh1.astype(jnp.float32) * wt1 + oh2.astype(jnp.float32) * wt2

        wcomm_ref[0] = wd
        for h in range(N_DEV - 1):
            wa = pltpu.make_async_remote_copy(
                src_ref=wcomm_ref.at[h], dst_ref=wcomm_ref.at[h + 1],
                send_sem=w_send.at[h], recv_sem=w_recv.at[h],
                device_id=(right,), device_id_type=pl.DeviceIdType.MESH,
            )
            wa.start()
            wa.wait()
        for s in range(N_DEV):
            org = lax.rem(my + N_DEV - s, N_DEV)
            wts_ref[pl.ds(org * T_PER, T_PER), :] = wcomm_ref[s]

        lane_t = lax.broadcasted_iota(jnp.int32, (T, E), 1)
        for e in range(E_PER):
            c1 = pltpu.make_async_copy(w1_ref.at[e], w1v_ref, load_sems.at[0])
            c2 = pltpu.make_async_copy(w2_ref.at[e], w2v_ref, load_sems.at[1])
            c1.start()
            c2.start()
            c1.wait()
            c2.wait()
            h1 = lax.dot_general(
                xg_ref[...], w1v_ref[...].astype(jnp.bfloat16), _DN,
                preferred_element_type=jnp.float32)
            hb = jnp.maximum(h1, 0.0).astype(jnp.bfloat16)
            p = lax.dot_general(
                hb, w2v_ref[...].astype(jnp.bfloat16), _DN,
                preferred_element_type=jnp.float32)
            ge = my * E_PER + e
            col = jnp.sum(
                jnp.where(lane_t == ge, wts_ref[...], 0.0),
                axis=1, keepdims=True)
            if e == 0:
                acc_ref[...] = p * col
            else:
                acc_ref[...] = acc_ref[...] + p * col

        idx0 = lax.rem(my + N_DEV - 1, N_DEV)
        rscomm_ref[0] = acc_ref[pl.ds(idx0 * T_PER, T_PER), :].astype(
            jnp.bfloat16)
        for h in range(N_DEV - 1):
            rr = pltpu.make_async_remote_copy(
                src_ref=rscomm_ref.at[h], dst_ref=rscomm_ref.at[h + 1],
                send_sem=rs_send.at[h], recv_sem=rs_recv.at[h],
                device_id=(right,), device_id_type=pl.DeviceIdType.MESH,
            )
            rr.start()
            rr.wait()
            idx = lax.rem(my + 2 * N_DEV - 2 - h, N_DEV)
            chunk = acc_ref[pl.ds(idx * T_PER, T_PER), :]
            if h < N_DEV - 2:
                rscomm_ref[h + 1] = (
                    rscomm_ref[h + 1].astype(jnp.float32) + chunk
                ).astype(jnp.bfloat16)
            else:
                out_ref[...] = rscomm_ref[h + 1].astype(jnp.float32) + chunk

    return pl.pallas_call(
        body,
        out_shape=jax.ShapeDtypeStruct((T_PER, D), jnp.float32),
        in_specs=[
            pl.BlockSpec(memory_space=pltpu.VMEM),
            pl.BlockSpec(memory_space=pltpu.VMEM),
            pl.BlockSpec(memory_space=pl.ANY),
            pl.BlockSpec(memory_space=pl.ANY),
        ],
        out_specs=pl.BlockSpec(memory_space=pltpu.VMEM),
        scratch_shapes=[
            pltpu.VMEM((N_DEV, D, E_PER), jnp.float32),
            pltpu.VMEM((T, E), jnp.float32),
            pltpu.VMEM((N_DEV, T_PER, E), jnp.float32),
            pltpu.VMEM((T, D), jnp.bfloat16),
            pltpu.VMEM((N_DEV, T_PER, D), jnp.bfloat16),
            pltpu.VMEM((T, D), jnp.float32),
            pltpu.VMEM((D, F), jnp.float32),
            pltpu.VMEM((F, D), jnp.float32),
            pltpu.VMEM((N_DEV, T_PER, D), jnp.bfloat16),
            pltpu.SemaphoreType.DMA((N_DEV - 1,)),
            pltpu.SemaphoreType.DMA((N_DEV - 1,)),
            pltpu.SemaphoreType.DMA((N_DEV - 1,)),
            pltpu.SemaphoreType.DMA((N_DEV - 1,)),
            pltpu.SemaphoreType.DMA((N_DEV - 1,)),
            pltpu.SemaphoreType.DMA((N_DEV - 1,)),
            pltpu.SemaphoreType.DMA((N_DEV - 1,)),
            pltpu.SemaphoreType.DMA((N_DEV - 1,)),
            pltpu.SemaphoreType.DMA((2,)),
        ],
        compiler_params=pltpu.CompilerParams(
            collective_id=0,
            vmem_limit_bytes=128 * 1024 * 1024,
        ),
    )(x, router, W1, W2)
